# baseline (speedup 1.0000x reference)
import functools

import jax
import jax.numpy as jnp
from jax import lax
from jax.experimental import pallas as pl
from jax.experimental.pallas import tpu as pltpu
from jax.experimental.pallas import tpu_sc as plsc

N = 10000
D = 128
HEADS = 8
OUT = D // HEADS
LAYERS = 4
E = 320000

NC = 2
NS = 16
NW = NC * NS
EPT = E // NW
B = 40
NCHUNK = EPT // B
NP = 10240
RPT = NP // NS

ROWS = 1000
NG = N // ROWS


def _ln(h, g, b, eps=1e-5):
    mu = jnp.mean(h, axis=-1, keepdims=True)
    var = jnp.mean((h - mu) ** 2, axis=-1, keepdims=True)
    return (h - mu) * jax.lax.rsqrt(var + eps) * g + b


def _tc_in_body(x_ref, win_ref, bin_ref, wg_ref, aw_ref,
                h_ref, xs_ref, ac_ref):
    h = jnp.dot(x_ref[...], win_ref[...], preferred_element_type=jnp.float32)
    h = h + bin_ref[...]
    h_ref[...] = h
    xh = jnp.dot(h, wg_ref[...], preferred_element_type=jnp.float32)
    ac = jnp.dot(xh, aw_ref[...], preferred_element_type=jnp.float32)
    xs_ref[...] = jnp.concatenate(
        [xh, ac, jnp.zeros((ROWS, 256 - D - 16), jnp.float32)], axis=1)
    ac_ref[...] = ac


def _head_expand_mat():
    j = lax.broadcasted_iota(jnp.int32, (16, D), 0)
    f = lax.broadcasted_iota(jnp.int32, (16, D), 1)
    return jnp.where(j == f // OUT, 1.0, 0.0).astype(jnp.float32)


def _merge(hprev, acc0, acc1, den0, den1, biasg, g, b):
    den = den0 + den1 + 1e-16
    dexp = jnp.dot(1.0 / den, _head_expand_mat(),
                   preferred_element_type=jnp.float32)
    out = (acc0 + acc1) * dexp + biasg
    return _ln(hprev + out, g, b)


def _tc_merge_body(hp_ref, a0_ref, a1_ref, d0_ref, d1_ref, bg_ref, g_ref, b_ref,
                   wg_ref, aw_ref,
                   h_ref, xs_ref, ac_ref):
    hn = _merge(hp_ref[...], a0_ref[...], a1_ref[...], d0_ref[...], d1_ref[...],
                bg_ref[...], g_ref[...], b_ref[...])
    h_ref[...] = hn
    xh = jnp.dot(hn, wg_ref[...], preferred_element_type=jnp.float32)
    ac = jnp.dot(xh, aw_ref[...], preferred_element_type=jnp.float32)
    xs_ref[...] = jnp.concatenate(
        [xh, ac, jnp.zeros((ROWS, 256 - D - 16), jnp.float32)], axis=1)
    ac_ref[...] = ac


def _tc_final_body(hp_ref, a0_ref, a1_ref, d0_ref, d1_ref, bg_ref, g_ref, b_ref,
                   wmu_ref, bmu_ref, wlv_ref, blv_ref,
                   h_ref, pooled_ref, mu_ref, lv_ref):
    i = pl.program_id(0)
    hn = _merge(hp_ref[...], a0_ref[...], a1_ref[...], d0_ref[...], d1_ref[...],
                bg_ref[...], g_ref[...], b_ref[...])
    h_ref[...] = hn
    part = jnp.sum(hn, axis=0, keepdims=True) * (1.0 / N)

    @pl.when(i == 0)
    def _():
        pooled_ref[...] = jnp.zeros_like(pooled_ref)

    pooled_ref[...] += part

    @pl.when(i == NG - 1)
    def _():
        pooled = pooled_ref[...]
        mu_ref[...] = jnp.dot(pooled, wmu_ref[...],
                              preferred_element_type=jnp.float32) + bmu_ref[...]
        lv_ref[...] = jnp.dot(pooled, wlv_ref[...],
                              preferred_element_type=jnp.float32) + blv_ref[...]


def _row_spec(cols):
    return pl.BlockSpec((ROWS, cols), lambda i: (i, 0))


def _full_spec(r, c):
    return pl.BlockSpec((r, c), lambda i: (0, 0))


_WSPEC = _full_spec(D, D)
_VSPEC = _full_spec(1, D)
_ASPEC = _full_spec(D, 16)


def _tc_in(x, win, bin_, wg, aw):
    return pl.pallas_call(
        _tc_in_body,
        grid=(NG,),
        in_specs=[_row_spec(D), _WSPEC, _VSPEC, _WSPEC, _ASPEC],
        out_specs=[_row_spec(D), _row_spec(256), _row_spec(16)],
        out_shape=[
            jax.ShapeDtypeStruct((N, D), jnp.float32),
            jax.ShapeDtypeStruct((N, 256), jnp.float32),
            jax.ShapeDtypeStruct((NP, 16), jnp.float32),
        ],
    )(x, win, bin_, wg, aw)


def _tc_merge(hp, a0, a1, d0, d1, bg, g, b, wg, aw):
    return pl.pallas_call(
        _tc_merge_body,
        grid=(NG,),
        in_specs=[_row_spec(D), _row_spec(D), _row_spec(D), _row_spec(16),
                  _row_spec(16), _VSPEC, _VSPEC, _VSPEC,
                  _WSPEC, _ASPEC],
        out_specs=[_row_spec(D), _row_spec(256), _row_spec(16)],
        out_shape=[
            jax.ShapeDtypeStruct((N, D), jnp.float32),
            jax.ShapeDtypeStruct((N, 256), jnp.float32),
            jax.ShapeDtypeStruct((NP, 16), jnp.float32),
        ],
    )(hp, a0, a1, d0, d1, bg, g, b, wg, aw)


def _tc_final(hp, a0, a1, d0, d1, bg, g, b, wmu, bmu, wlv, blv):
    return pl.pallas_call(
        _tc_final_body,
        grid=(NG,),
        in_specs=[_row_spec(D), _row_spec(D), _row_spec(D), _row_spec(16),
                  _row_spec(16), _VSPEC, _VSPEC, _VSPEC,
                  _WSPEC, _VSPEC, _WSPEC, _VSPEC],
        out_specs=[_row_spec(D), _full_spec(1, D), _full_spec(1, D),
                   _full_spec(1, D)],
        out_shape=[
            jax.ShapeDtypeStruct((N, D), jnp.float32),
            jax.ShapeDtypeStruct((1, D), jnp.float32),
            jax.ShapeDtypeStruct((1, D), jnp.float32),
            jax.ShapeDtypeStruct((1, D), jnp.float32),
        ],
    )(hp, a0, a1, d0, d1, bg, g, b, wmu, bmu, wlv, blv)


def _lane_gather(v, idx16):
    dn = jax.lax.GatherDimensionNumbers(
        offset_dims=(), collapsed_slice_dims=(0,), start_index_map=(0,))
    return jax.lax.gather(v, idx16.reshape(16, 1), dn, (1,),
                          mode=jax.lax.GatherScatterMode.PROMISE_IN_BOUNDS)


_SC_MESH = plsc.VectorSubcoreMesh(core_axis_name="c", subcore_axis_name="s")

NP8 = NP // 8


@functools.partial(
    pl.kernel,
    mesh=_SC_MESH,
    out_type=(
        jax.ShapeDtypeStruct((NC * NP, D), jnp.float32),
        jax.ShapeDtypeStruct((NC * NP8, D), jnp.float32),
    ),
    scratch_types=[
        pltpu.VMEM((B,), jnp.int32),
        pltpu.VMEM((B,), jnp.int32),
        pltpu.VMEM((B,), jnp.int32),
        pltpu.VMEM((B,), jnp.int32),
        pltpu.VMEM((B,), jnp.int32),
        pltpu.VMEM((B,), jnp.int32),
        pltpu.VMEM((B, 256), jnp.float32),
        pltpu.VMEM((B, 256), jnp.float32),
        pltpu.VMEM((B, D), jnp.float32),
        pltpu.VMEM((B, D), jnp.float32),
        pltpu.VMEM((B, D), jnp.float32),
        pltpu.VMEM((8, D), jnp.float32),
        pltpu.VMEM_SHARED((NP, D), jnp.float32),
        pltpu.VMEM_SHARED((NP8, D), jnp.float32),
        pltpu.SemaphoreType.DMA,
        pltpu.SemaphoreType.DMA,
        pltpu.SemaphoreType.DMA,
    ],
)
def _sc_edge(src_hbm, dst_hbm, ac8_hbm, xs_hbm,
             acc_out, den_out,
             srcv0, srcv1, dstv0, dstv1, d8v0, d8v1, xrows0, xrows1,
             msgbuf, drows, exd, zbuf,
             acc_sh, den_sh, semx0, semx1, sem2):
    c = lax.axis_index("c")
    s = lax.axis_index("s")
    wid = c * NS + s
    r0 = s * RPT
    r8 = s * (NP8 // NS)

    bufs = ((srcv0, dstv0, d8v0, xrows0, semx0),
            (srcv1, dstv1, d8v1, xrows1, semx1))

    zv = jnp.zeros((16,), jnp.float32)

    def zrow(r, carry):
        for cc in range(D // 16):
            zbuf[r, pl.ds(cc * 16, 16)] = zv
        return carry

    lax.fori_loop(0, 8, zrow, 0)

    for j in range(RPT // 8):
        pltpu.sync_copy(zbuf, acc_sh.at[pl.ds(r0 + j * 8, 8)])
    for j in range(NP8 // NS // 8):
        pltpu.sync_copy(zbuf, den_sh.at[pl.ds(r8 + j * 8, 8)])
    plsc.subcore_barrier()

    def zexd(r, carry):
        for cc in range(D // 16):
            exd[r, pl.ds(cc * 16, 16)] = zv
        return carry

    lax.fori_loop(0, B, zexd, 0)

    ebase = wid * EPT
    lane = lax.iota(jnp.int32, 16)
    perm_hi = (lane & 7) + 8
    GROUPS = ((0, tuple(range(16))), (16, tuple(range(16))),
              (24, tuple(range(8, 16))))

    def load_idx(k, sv, dv, d8):
        base = ebase + k * B
        pltpu.sync_copy(src_hbm.at[pl.ds(base, B)], sv)
        pltpu.sync_copy(dst_hbm.at[pl.ds(base, B)], dv)
        for off in (0, 16, 24):
            d8[pl.ds(off, 16)] = lax.shift_right_logical(dv[pl.ds(off, 16)], 3)

    load_idx(0, srcv0, dstv0, d8v0)
    pltpu.async_copy(xs_hbm.at[srcv0], xrows0, semx0)

    def body(kk, carry):
        for b in (0, 1):
            k = kk * 2 + b
            sv, dv, d8, xr, semx = bufs[b]
            nsv, ndv, nd8, nxr, nsemx = bufs[1 - b]

            @pl.when(k + 1 < NCHUNK)
            def _():
                load_idx(k + 1, nsv, ndv, nd8)
                pltpu.async_copy(xs_hbm.at[nsv], nxr, nsemx)

            cp_d = pltpu.async_copy(ac8_hbm.at[d8], drows, sem2)
            pltpu.make_async_copy(xs_hbm.at[sv], xr, semx).wait()
            cp_d.wait()

            for gg, jr in GROUPS:
                dvv = dv[pl.ds(gg, 16)]
                doffs = lax.shift_left(dvv & 7, 4)
                for j in jr:
                    i = gg + j
                    doff = doffs[j]
                    arow = xr[i, pl.ds(D, 16)]
                    drow = drows[i, pl.ds(doff, 16)]
                    t = arow + _lane_gather(drow, perm_hi)
                    t = jnp.maximum(t, t * 0.2)
                    ex = jnp.exp(t)
                    exd[i, pl.ds(doff, 16)] = ex
                    for hh in range(HEADS):
                        msgbuf[i, pl.ds(hh * OUT, OUT)] = \
                            xr[i, pl.ds(hh * OUT, OUT)] * ex[hh]

            pltpu.sync_copy(msgbuf, acc_sh.at[dv], add=True)
            pltpu.sync_copy(exd, den_sh.at[d8], add=True)

            for gg, jr in GROUPS:
                dvv = dv[pl.ds(gg, 16)]
                doffs = lax.shift_left(dvv & 7, 4)
                for j in jr:
                    exd[gg + j, pl.ds(doffs[j], 16)] = zv
        return carry

    lax.fori_loop(0, NCHUNK // 2, body, 0)
    plsc.subcore_barrier()

    o0 = c * NP + r0
    for j in range(RPT // 8):
        pltpu.sync_copy(acc_sh.at[pl.ds(r0 + j * 8, 8)], zbuf)
        pltpu.sync_copy(zbuf, acc_out.at[pl.ds(o0 + j * 8, 8)])
    o8 = c * NP8 + r8
    for j in range(NP8 // NS // 8):
        pltpu.sync_copy(den_sh.at[pl.ds(r8 + j * 8, 8)], zbuf)
        pltpu.sync_copy(zbuf, den_out.at[pl.ds(o8 + j * 8, 8)])


def _att_mat(att_src, att_dst):
    f = jnp.arange(D)
    j = jnp.arange(16)
    mask = (f[:, None] // OUT) == (j[None, :] % HEADS)
    val = jnp.where(j[None, :] < HEADS, att_src.reshape(-1)[:, None],
                    att_dst.reshape(-1)[:, None])
    return jnp.where(mask, val, 0.0).astype(jnp.float32)


def kernel(x, edge_index, params):
    src = edge_index[0]
    dst = edge_index[1]

    att = [_att_mat(params['gat'][l]['att_src'], params['gat'][l]['att_dst'])
           for l in range(LAYERS)]
    row = lambda v: v.reshape(1, D)

    h, xs, ac = _tc_in(
        x, params['W_in'], row(params['b_in']),
        params['gat'][0]['W'], att[0])

    mu = lv = pooled = None
    for l in range(LAYERS):
        ac8 = ac.reshape(NP8, D)
        accf, denf = _sc_edge(src, dst, ac8, xs)
        acc = accf.reshape(NC, NP, D)
        den = denf.reshape(NC, NP, 16)
        gp = params['gat'][l]
        lp = params['ln'][l]
        args = (h, acc[0], acc[1], den[0], den[1],
                row(gp['bias']), row(lp['g']), row(lp['b']))
        if l < LAYERS - 1:
            h, xs, ac = _tc_merge(
                *args, params['gat'][l + 1]['W'], att[l + 1])
        else:
            h, pooled, mu, lv = _tc_final(
                *args, params['W_mu'], row(params['b_mu']),
                params['W_lv'], row(params['b_lv']))

    return (mu, lv, h, pooled)


# speedup vs baseline: 1.1150x; 1.1150x over previous
import functools

import jax
import jax.numpy as jnp
from jax import lax
from jax.experimental import pallas as pl
from jax.experimental.pallas import tpu as pltpu
from jax.experimental.pallas import tpu_sc as plsc

N = 10000
D = 128
HEADS = 8
OUT = D // HEADS
LAYERS = 4
E = 320000

NC = 2
NS = 16
NW = NC * NS
EPT = E // NW
B = 40
NCHUNK = EPT // B
NP = 10240
RPT = NP // NS

ROWS = 1000
NG = N // ROWS


def _ln(h, g, b, eps=1e-5):
    mu = jnp.mean(h, axis=-1, keepdims=True)
    var = jnp.mean((h - mu) ** 2, axis=-1, keepdims=True)
    return (h - mu) * jax.lax.rsqrt(var + eps) * g + b


def _tc_in_body(x_ref, win_ref, bin_ref, wg_ref, aw_ref,
                h_ref, xs_ref, ac_ref):
    h = jnp.dot(x_ref[...], win_ref[...], preferred_element_type=jnp.float32)
    h = h + bin_ref[...]
    h_ref[...] = h
    xh = jnp.dot(h, wg_ref[...], preferred_element_type=jnp.float32)
    ac = jnp.dot(xh, aw_ref[...], preferred_element_type=jnp.float32)
    xs_ref[...] = jnp.concatenate(
        [xh, ac, jnp.zeros((ROWS, 256 - D - 16), jnp.float32)], axis=1)
    ac_ref[...] = ac


def _head_expand_mat():
    j = lax.broadcasted_iota(jnp.int32, (16, D), 0)
    f = lax.broadcasted_iota(jnp.int32, (16, D), 1)
    return jnp.where(j == f // OUT, 1.0, 0.0).astype(jnp.float32)


def _merge(hprev, acc0, acc1, den0, den1, biasg, g, b):
    den = den0 + den1 + 1e-16
    dexp = jnp.dot(1.0 / den, _head_expand_mat(),
                   preferred_element_type=jnp.float32)
    out = (acc0 + acc1) * dexp + biasg
    return _ln(hprev + out, g, b)


def _tc_merge_body(hp_ref, a0_ref, a1_ref, d0_ref, d1_ref, bg_ref, g_ref, b_ref,
                   wg_ref, aw_ref,
                   h_ref, xs_ref, ac_ref):
    hn = _merge(hp_ref[...], a0_ref[...], a1_ref[...], d0_ref[...], d1_ref[...],
                bg_ref[...], g_ref[...], b_ref[...])
    h_ref[...] = hn
    xh = jnp.dot(hn, wg_ref[...], preferred_element_type=jnp.float32)
    ac = jnp.dot(xh, aw_ref[...], preferred_element_type=jnp.float32)
    xs_ref[...] = jnp.concatenate(
        [xh, ac, jnp.zeros((ROWS, 256 - D - 16), jnp.float32)], axis=1)
    ac_ref[...] = ac


def _tc_final_body(hp_ref, a0_ref, a1_ref, d0_ref, d1_ref, bg_ref, g_ref, b_ref,
                   wmu_ref, bmu_ref, wlv_ref, blv_ref,
                   h_ref, pooled_ref, mu_ref, lv_ref):
    i = pl.program_id(0)
    hn = _merge(hp_ref[...], a0_ref[...], a1_ref[...], d0_ref[...], d1_ref[...],
                bg_ref[...], g_ref[...], b_ref[...])
    h_ref[...] = hn
    part = jnp.sum(hn, axis=0, keepdims=True) * (1.0 / N)

    @pl.when(i == 0)
    def _():
        pooled_ref[...] = jnp.zeros_like(pooled_ref)

    pooled_ref[...] += part

    @pl.when(i == NG - 1)
    def _():
        pooled = pooled_ref[...]
        mu_ref[...] = jnp.dot(pooled, wmu_ref[...],
                              preferred_element_type=jnp.float32) + bmu_ref[...]
        lv_ref[...] = jnp.dot(pooled, wlv_ref[...],
                              preferred_element_type=jnp.float32) + blv_ref[...]


def _row_spec(cols):
    return pl.BlockSpec((ROWS, cols), lambda i: (i, 0))


def _full_spec(r, c):
    return pl.BlockSpec((r, c), lambda i: (0, 0))


_WSPEC = _full_spec(D, D)
_VSPEC = _full_spec(1, D)
_ASPEC = _full_spec(D, 16)


def _tc_in(x, win, bin_, wg, aw):
    return pl.pallas_call(
        _tc_in_body,
        grid=(NG,),
        in_specs=[_row_spec(D), _WSPEC, _VSPEC, _WSPEC, _ASPEC],
        out_specs=[_row_spec(D), _row_spec(256), _row_spec(16)],
        out_shape=[
            jax.ShapeDtypeStruct((N, D), jnp.float32),
            jax.ShapeDtypeStruct((N, 256), jnp.float32),
            jax.ShapeDtypeStruct((NP, 16), jnp.float32),
        ],
    )(x, win, bin_, wg, aw)


def _tc_merge(hp, a0, a1, d0, d1, bg, g, b, wg, aw):
    return pl.pallas_call(
        _tc_merge_body,
        grid=(NG,),
        in_specs=[_row_spec(D), _row_spec(D), _row_spec(D), _row_spec(16),
                  _row_spec(16), _VSPEC, _VSPEC, _VSPEC,
                  _WSPEC, _ASPEC],
        out_specs=[_row_spec(D), _row_spec(256), _row_spec(16)],
        out_shape=[
            jax.ShapeDtypeStruct((N, D), jnp.float32),
            jax.ShapeDtypeStruct((N, 256), jnp.float32),
            jax.ShapeDtypeStruct((NP, 16), jnp.float32),
        ],
    )(hp, a0, a1, d0, d1, bg, g, b, wg, aw)


def _tc_final(hp, a0, a1, d0, d1, bg, g, b, wmu, bmu, wlv, blv):
    return pl.pallas_call(
        _tc_final_body,
        grid=(NG,),
        in_specs=[_row_spec(D), _row_spec(D), _row_spec(D), _row_spec(16),
                  _row_spec(16), _VSPEC, _VSPEC, _VSPEC,
                  _WSPEC, _VSPEC, _WSPEC, _VSPEC],
        out_specs=[_row_spec(D), _full_spec(1, D), _full_spec(1, D),
                   _full_spec(1, D)],
        out_shape=[
            jax.ShapeDtypeStruct((N, D), jnp.float32),
            jax.ShapeDtypeStruct((1, D), jnp.float32),
            jax.ShapeDtypeStruct((1, D), jnp.float32),
            jax.ShapeDtypeStruct((1, D), jnp.float32),
        ],
    )(hp, a0, a1, d0, d1, bg, g, b, wmu, bmu, wlv, blv)


def _lane_gather(v, idx16):
    dn = jax.lax.GatherDimensionNumbers(
        offset_dims=(), collapsed_slice_dims=(0,), start_index_map=(0,))
    return jax.lax.gather(v, idx16.reshape(16, 1), dn, (1,),
                          mode=jax.lax.GatherScatterMode.PROMISE_IN_BOUNDS)


_SC_MESH = plsc.VectorSubcoreMesh(core_axis_name="c", subcore_axis_name="s")

NP8 = NP // 8


@functools.partial(
    pl.kernel,
    mesh=_SC_MESH,
    out_type=(
        jax.ShapeDtypeStruct((NC * NP, D), jnp.float32),
        jax.ShapeDtypeStruct((NC * NP8, D), jnp.float32),
    ),
    scratch_types=[
        pltpu.VMEM((B,), jnp.int32),
        pltpu.VMEM((B,), jnp.int32),
        pltpu.VMEM((B,), jnp.int32),
        pltpu.VMEM((B,), jnp.int32),
        pltpu.VMEM((B, 256), jnp.float32),
        pltpu.VMEM((B, D), jnp.float32),
        pltpu.VMEM((B, D), jnp.float32),
        pltpu.VMEM((B, D), jnp.float32),
        pltpu.VMEM((8, D), jnp.float32),
        pltpu.SemaphoreType.DMA,
        pltpu.SemaphoreType.DMA,
        pltpu.VMEM_SHARED((NP, D), jnp.float32),
        pltpu.VMEM_SHARED((NP8, D), jnp.float32),
    ],
)
def _sc_edge(src_hbm, dst_hbm, ac8_hbm, xs_hbm,
             acc_out, den_out,
             srcv, dstv, d8v, doffv, xrows, msgbuf, drows, exd, zbuf,
             sem1, sem2, acc_sh, den_sh):
    c = lax.axis_index("c")
    s = lax.axis_index("s")
    wid = c * NS + s
    r0 = s * RPT
    r8 = s * (NP8 // NS)

    zv = jnp.zeros((16,), jnp.float32)

    def zrow(r, carry):
        for cc in range(D // 16):
            zbuf[r, pl.ds(cc * 16, 16)] = zv
        return carry

    lax.fori_loop(0, 8, zrow, 0)

    for j in range(RPT // 8):
        pltpu.sync_copy(zbuf, acc_sh.at[pl.ds(r0 + j * 8, 8)])
    for j in range(NP8 // NS // 8):
        pltpu.sync_copy(zbuf, den_sh.at[pl.ds(r8 + j * 8, 8)])
    plsc.subcore_barrier()

    def zexd(r, carry):
        for cc in range(D // 16):
            exd[r, pl.ds(cc * 16, 16)] = zv
        return carry

    lax.fori_loop(0, B, zexd, 0)

    ebase = wid * EPT
    lane = lax.iota(jnp.int32, 16)
    perm_hi = (lane & 7) + 8
    GROUPS = ((0, tuple(range(16))), (16, tuple(range(16))),
              (24, tuple(range(8, 16))))

    def chunk(k, carry):
        base = ebase + k * B
        pltpu.sync_copy(src_hbm.at[pl.ds(base, B)], srcv)
        pltpu.sync_copy(dst_hbm.at[pl.ds(base, B)], dstv)

        for off in (0, 16, 24):
            dv = dstv[pl.ds(off, 16)]
            d8v[pl.ds(off, 16)] = lax.shift_right_logical(dv, 3)
            doffv[pl.ds(off, 16)] = lax.shift_left(dv & 7, 4)

        cp_x = pltpu.async_copy(xs_hbm.at[srcv], xrows, sem1)
        pltpu.async_copy(ac8_hbm.at[d8v], drows, sem2).wait()
        cp_x.wait()

        for gg, jr in GROUPS:
            dvv = doffv[pl.ds(gg, 16)]
            for j in jr:
                i = gg + j
                doff = dvv[j]
                arow = xrows[i, pl.ds(D, 16)]
                drow = drows[i, pl.ds(doff, 16)]
                t = arow + _lane_gather(drow, perm_hi)
                t = jnp.maximum(t, t * 0.2)
                ex = jnp.exp(t)
                exd[i, pl.ds(doff, 16)] = ex
                for hh in range(HEADS):
                    bc = _lane_gather(ex, lane * 0 + hh)
                    msgbuf[i, pl.ds(hh * OUT, OUT)] = \
                        xrows[i, pl.ds(hh * OUT, OUT)] * bc

        pltpu.sync_copy(msgbuf, acc_sh.at[dstv], add=True)
        pltpu.sync_copy(exd, den_sh.at[d8v], add=True)

        for gg, jr in GROUPS:
            dvv = doffv[pl.ds(gg, 16)]
            for j in jr:
                exd[gg + j, pl.ds(dvv[j], 16)] = zv
        return carry

    lax.fori_loop(0, NCHUNK, chunk, 0)
    plsc.subcore_barrier()

    o0 = c * NP + r0
    for j in range(RPT // 8):
        pltpu.sync_copy(acc_sh.at[pl.ds(r0 + j * 8, 8)], zbuf)
        pltpu.sync_copy(zbuf, acc_out.at[pl.ds(o0 + j * 8, 8)])
    o8 = c * NP8 + r8
    for j in range(NP8 // NS // 8):
        pltpu.sync_copy(den_sh.at[pl.ds(r8 + j * 8, 8)], zbuf)
        pltpu.sync_copy(zbuf, den_out.at[pl.ds(o8 + j * 8, 8)])


def _att_mat(att_src, att_dst):
    f = jnp.arange(D)
    j = jnp.arange(16)
    mask = (f[:, None] // OUT) == (j[None, :] % HEADS)
    val = jnp.where(j[None, :] < HEADS, att_src.reshape(-1)[:, None],
                    att_dst.reshape(-1)[:, None])
    return jnp.where(mask, val, 0.0).astype(jnp.float32)


def kernel(x, edge_index, params):
    src = edge_index[0]
    dst = edge_index[1]

    att = [_att_mat(params['gat'][l]['att_src'], params['gat'][l]['att_dst'])
           for l in range(LAYERS)]
    row = lambda v: v.reshape(1, D)

    h, xs, ac = _tc_in(
        x, params['W_in'], row(params['b_in']),
        params['gat'][0]['W'], att[0])

    mu = lv = pooled = None
    for l in range(LAYERS):
        ac8 = ac.reshape(NP8, D)
        accf, denf = _sc_edge(src, dst, ac8, xs)
        acc = accf.reshape(NC, NP, D)
        den = denf.reshape(NC, NP, 16)
        gp = params['gat'][l]
        lp = params['ln'][l]
        args = (h, acc[0], acc[1], den[0], den[1],
                row(gp['bias']), row(lp['g']), row(lp['b']))
        if l < LAYERS - 1:
            h, xs, ac = _tc_merge(
                *args, params['gat'][l + 1]['W'], att[l + 1])
        else:
            h, pooled, mu, lv = _tc_final(
                *args, params['W_mu'], row(params['b_mu']),
                params['W_lv'], row(params['b_lv']))

    return (mu, lv, h, pooled)


# speedup vs baseline: 1.1154x; 1.0004x over previous
import functools

import jax
import jax.numpy as jnp
from jax import lax
from jax.experimental import pallas as pl
from jax.experimental.pallas import tpu as pltpu
from jax.experimental.pallas import tpu_sc as plsc

N = 10000
D = 128
HEADS = 8
OUT = D // HEADS
LAYERS = 4
E = 320000

NC = 2
NS = 16
NW = NC * NS
EPT = E // NW
B = 40
NCHUNK = EPT // B
NP = 10240
RPT = NP // NS

ROWS = 1000
NG = N // ROWS


def _ln(h, g, b, eps=1e-5):
    mu = jnp.mean(h, axis=-1, keepdims=True)
    var = jnp.mean((h - mu) ** 2, axis=-1, keepdims=True)
    return (h - mu) * jax.lax.rsqrt(var + eps) * g + b


def _tc_in_body(x_ref, win_ref, bin_ref, wg_ref, aw_ref,
                h_ref, xs_ref, ac_ref):
    h = jnp.dot(x_ref[...], win_ref[...], preferred_element_type=jnp.float32)
    h = h + bin_ref[...]
    h_ref[...] = h
    xh = jnp.dot(h, wg_ref[...], preferred_element_type=jnp.float32)
    ac = jnp.dot(xh, aw_ref[...], preferred_element_type=jnp.float32)
    xs_ref[...] = jnp.concatenate(
        [xh, ac, jnp.zeros((ROWS, 256 - D - 16), jnp.float32)], axis=1)
    ac_ref[...] = ac


def _head_expand_mat():
    j = lax.broadcasted_iota(jnp.int32, (16, D), 0)
    f = lax.broadcasted_iota(jnp.int32, (16, D), 1)
    return jnp.where(j == f // OUT, 1.0, 0.0).astype(jnp.float32)


def _merge(hprev, acc0, acc1, den0, den1, biasg, g, b):
    den = den0 + den1 + 1e-16
    dexp = jnp.dot(1.0 / den, _head_expand_mat(),
                   preferred_element_type=jnp.float32)
    out = (acc0 + acc1) * dexp + biasg
    return _ln(hprev + out, g, b)


def _tc_merge_body(hp_ref, a0_ref, a1_ref, d0_ref, d1_ref, bg_ref, g_ref, b_ref,
                   wg_ref, aw_ref,
                   h_ref, xs_ref, ac_ref):
    hn = _merge(hp_ref[...], a0_ref[...], a1_ref[...], d0_ref[...], d1_ref[...],
                bg_ref[...], g_ref[...], b_ref[...])
    h_ref[...] = hn
    xh = jnp.dot(hn, wg_ref[...], preferred_element_type=jnp.float32)
    ac = jnp.dot(xh, aw_ref[...], preferred_element_type=jnp.float32)
    xs_ref[...] = jnp.concatenate(
        [xh, ac, jnp.zeros((ROWS, 256 - D - 16), jnp.float32)], axis=1)
    ac_ref[...] = ac


def _tc_final_body(hp_ref, a0_ref, a1_ref, d0_ref, d1_ref, bg_ref, g_ref, b_ref,
                   wmu_ref, bmu_ref, wlv_ref, blv_ref,
                   h_ref, pooled_ref, mu_ref, lv_ref):
    i = pl.program_id(0)
    hn = _merge(hp_ref[...], a0_ref[...], a1_ref[...], d0_ref[...], d1_ref[...],
                bg_ref[...], g_ref[...], b_ref[...])
    h_ref[...] = hn
    part = jnp.sum(hn, axis=0, keepdims=True) * (1.0 / N)

    @pl.when(i == 0)
    def _():
        pooled_ref[...] = jnp.zeros_like(pooled_ref)

    pooled_ref[...] += part

    @pl.when(i == NG - 1)
    def _():
        pooled = pooled_ref[...]
        mu_ref[...] = jnp.dot(pooled, wmu_ref[...],
                              preferred_element_type=jnp.float32) + bmu_ref[...]
        lv_ref[...] = jnp.dot(pooled, wlv_ref[...],
                              preferred_element_type=jnp.float32) + blv_ref[...]


def _row_spec(cols):
    return pl.BlockSpec((ROWS, cols), lambda i: (i, 0))


def _full_spec(r, c):
    return pl.BlockSpec((r, c), lambda i: (0, 0))


_WSPEC = _full_spec(D, D)
_VSPEC = _full_spec(1, D)
_ASPEC = _full_spec(D, 16)


def _tc_in(x, win, bin_, wg, aw):
    return pl.pallas_call(
        _tc_in_body,
        grid=(NG,),
        in_specs=[_row_spec(D), _WSPEC, _VSPEC, _WSPEC, _ASPEC],
        out_specs=[_row_spec(D), _row_spec(256), _row_spec(16)],
        out_shape=[
            jax.ShapeDtypeStruct((N, D), jnp.float32),
            jax.ShapeDtypeStruct((N, 256), jnp.float32),
            jax.ShapeDtypeStruct((NP, 16), jnp.float32),
        ],
    )(x, win, bin_, wg, aw)


def _tc_merge(hp, a0, a1, d0, d1, bg, g, b, wg, aw):
    return pl.pallas_call(
        _tc_merge_body,
        grid=(NG,),
        in_specs=[_row_spec(D), _row_spec(D), _row_spec(D), _row_spec(16),
                  _row_spec(16), _VSPEC, _VSPEC, _VSPEC,
                  _WSPEC, _ASPEC],
        out_specs=[_row_spec(D), _row_spec(256), _row_spec(16)],
        out_shape=[
            jax.ShapeDtypeStruct((N, D), jnp.float32),
            jax.ShapeDtypeStruct((N, 256), jnp.float32),
            jax.ShapeDtypeStruct((NP, 16), jnp.float32),
        ],
    )(hp, a0, a1, d0, d1, bg, g, b, wg, aw)


def _tc_final(hp, a0, a1, d0, d1, bg, g, b, wmu, bmu, wlv, blv):
    return pl.pallas_call(
        _tc_final_body,
        grid=(NG,),
        in_specs=[_row_spec(D), _row_spec(D), _row_spec(D), _row_spec(16),
                  _row_spec(16), _VSPEC, _VSPEC, _VSPEC,
                  _WSPEC, _VSPEC, _WSPEC, _VSPEC],
        out_specs=[_row_spec(D), _full_spec(1, D), _full_spec(1, D),
                   _full_spec(1, D)],
        out_shape=[
            jax.ShapeDtypeStruct((N, D), jnp.float32),
            jax.ShapeDtypeStruct((1, D), jnp.float32),
            jax.ShapeDtypeStruct((1, D), jnp.float32),
            jax.ShapeDtypeStruct((1, D), jnp.float32),
        ],
    )(hp, a0, a1, d0, d1, bg, g, b, wmu, bmu, wlv, blv)


def _lane_gather(v, idx16):
    dn = jax.lax.GatherDimensionNumbers(
        offset_dims=(), collapsed_slice_dims=(0,), start_index_map=(0,))
    return jax.lax.gather(v, idx16.reshape(16, 1), dn, (1,),
                          mode=jax.lax.GatherScatterMode.PROMISE_IN_BOUNDS)


_SC_MESH = plsc.VectorSubcoreMesh(core_axis_name="c", subcore_axis_name="s")

NP8 = NP // 8


@functools.partial(
    pl.kernel,
    mesh=_SC_MESH,
    out_type=(
        jax.ShapeDtypeStruct((NC * NP, D), jnp.float32),
        jax.ShapeDtypeStruct((NC * NP8, D), jnp.float32),
    ),
    scratch_types=[
        pltpu.VMEM((B,), jnp.int32),
        pltpu.VMEM((B,), jnp.int32),
        pltpu.VMEM((B,), jnp.int32),
        pltpu.VMEM((B,), jnp.int32),
        pltpu.VMEM((B, 256), jnp.float32),
        pltpu.VMEM((B, D), jnp.float32),
        pltpu.VMEM((B, D), jnp.float32),
        pltpu.VMEM((B, D), jnp.float32),
        pltpu.VMEM((8, D), jnp.float32),
        pltpu.SemaphoreType.DMA,
        pltpu.SemaphoreType.DMA,
        pltpu.VMEM_SHARED((NP, D), jnp.float32),
        pltpu.VMEM_SHARED((NP8, D), jnp.float32),
    ],
)
def _sc_edge(src_hbm, dst_hbm, ac8_hbm, xs_hbm,
             acc_out, den_out,
             srcv, dstv, d8v, doffv, xrows, msgbuf, drows, exd, zbuf,
             sem1, sem2, acc_sh, den_sh):
    c = lax.axis_index("c")
    s = lax.axis_index("s")
    wid = c * NS + s
    r0 = s * RPT
    r8 = s * (NP8 // NS)

    zv = jnp.zeros((16,), jnp.float32)

    def zrow(r, carry):
        for cc in range(D // 16):
            zbuf[r, pl.ds(cc * 16, 16)] = zv
        return carry

    lax.fori_loop(0, 8, zrow, 0)

    for j in range(RPT // 8):
        pltpu.sync_copy(zbuf, acc_sh.at[pl.ds(r0 + j * 8, 8)])
    for j in range(NP8 // NS // 8):
        pltpu.sync_copy(zbuf, den_sh.at[pl.ds(r8 + j * 8, 8)])
    plsc.subcore_barrier()

    def zexd(r, carry):
        for cc in range(D // 16):
            exd[r, pl.ds(cc * 16, 16)] = zv
        return carry

    lax.fori_loop(0, B, zexd, 0)

    ebase = wid * EPT
    lane = lax.iota(jnp.int32, 16)
    perm_hi = (lane & 7) + 8
    GROUPS = ((0, tuple(range(16))), (16, tuple(range(16))),
              (24, tuple(range(8, 16))))

    def chunk(k, carry):
        base = ebase + k * B
        pltpu.sync_copy(src_hbm.at[pl.ds(base, B)], srcv)
        pltpu.sync_copy(dst_hbm.at[pl.ds(base, B)], dstv)

        for off in (0, 16, 24):
            dv = dstv[pl.ds(off, 16)]
            d8v[pl.ds(off, 16)] = lax.shift_right_logical(dv, 3)
            doffv[pl.ds(off, 16)] = lax.shift_left(dv & 7, 4)

        cp_x = pltpu.async_copy(xs_hbm.at[srcv], xrows, sem1)
        pltpu.async_copy(ac8_hbm.at[d8v], drows, sem2).wait()
        cp_x.wait()

        for gg, jr in GROUPS:
            dvv = doffv[pl.ds(gg, 16)]
            for j in jr:
                i = gg + j
                doff = dvv[j]
                arow = xrows[i, pl.ds(D, 16)]
                drow = drows[i, pl.ds(doff, 16)]
                t = arow + _lane_gather(drow, perm_hi)
                t = jnp.maximum(t, t * 0.2)
                ex = jnp.exp(t)
                exd[i, pl.ds(doff, 16)] = ex
                for hh in range(HEADS):
                    msgbuf[i, pl.ds(hh * OUT, OUT)] = \
                        xrows[i, pl.ds(hh * OUT, OUT)] * ex[hh]

        pltpu.sync_copy(msgbuf, acc_sh.at[dstv], add=True)
        pltpu.sync_copy(exd, den_sh.at[d8v], add=True)

        for gg, jr in GROUPS:
            dvv = doffv[pl.ds(gg, 16)]
            for j in jr:
                exd[gg + j, pl.ds(dvv[j], 16)] = zv
        return carry

    lax.fori_loop(0, NCHUNK, chunk, 0)
    plsc.subcore_barrier()

    o0 = c * NP + r0
    for j in range(RPT // 8):
        pltpu.sync_copy(acc_sh.at[pl.ds(r0 + j * 8, 8)], zbuf)
        pltpu.sync_copy(zbuf, acc_out.at[pl.ds(o0 + j * 8, 8)])
    o8 = c * NP8 + r8
    for j in range(NP8 // NS // 8):
        pltpu.sync_copy(den_sh.at[pl.ds(r8 + j * 8, 8)], zbuf)
        pltpu.sync_copy(zbuf, den_out.at[pl.ds(o8 + j * 8, 8)])


def _att_mat(att_src, att_dst):
    f = jnp.arange(D)
    j = jnp.arange(16)
    mask = (f[:, None] // OUT) == (j[None, :] % HEADS)
    val = jnp.where(j[None, :] < HEADS, att_src.reshape(-1)[:, None],
                    att_dst.reshape(-1)[:, None])
    return jnp.where(mask, val, 0.0).astype(jnp.float32)


def kernel(x, edge_index, params):
    src = edge_index[0]
    dst = edge_index[1]

    att = [_att_mat(params['gat'][l]['att_src'], params['gat'][l]['att_dst'])
           for l in range(LAYERS)]
    row = lambda v: v.reshape(1, D)

    h, xs, ac = _tc_in(
        x, params['W_in'], row(params['b_in']),
        params['gat'][0]['W'], att[0])

    mu = lv = pooled = None
    for l in range(LAYERS):
        ac8 = ac.reshape(NP8, D)
        accf, denf = _sc_edge(src, dst, ac8, xs)
        acc = accf.reshape(NC, NP, D)
        den = denf.reshape(NC, NP, 16)
        gp = params['gat'][l]
        lp = params['ln'][l]
        args = (h, acc[0], acc[1], den[0], den[1],
                row(gp['bias']), row(lp['g']), row(lp['b']))
        if l < LAYERS - 1:
            h, xs, ac = _tc_merge(
                *args, params['gat'][l + 1]['W'], att[l + 1])
        else:
            h, pooled, mu, lv = _tc_final(
                *args, params['W_mu'], row(params['b_mu']),
                params['W_lv'], row(params['b_lv']))

    return (mu, lv, h, pooled)
